# baseline (speedup 1.0000x reference)
import functools

import jax
import jax.numpy as jnp
from jax import lax
from jax.experimental import pallas as pl
from jax.experimental.pallas import tpu as pltpu
from jax.experimental.pallas import tpu_sc as plsc

N = 10000
E = 320000
D = 128
NC = 2
NS = 16
NW = NC * NS
EPW = E // NW
CHUNK = 80
NCH = EPW // CHUNK
NP = 10240
RPT = NP // NS

_mesh = plsc.VectorSubcoreMesh(core_axis_name="c", subcore_axis_name="s")


@functools.partial(
    pl.kernel,
    out_type=jax.ShapeDtypeStruct((NC, NP, D), jnp.float32),
    mesh=_mesh,
    scratch_types=[
        pltpu.VMEM((NCH, CHUNK), jnp.int32),
        pltpu.VMEM((CHUNK, D), jnp.float32),
        pltpu.VMEM_SHARED((NP, D), jnp.float32),
    ],
)
def _deg_kernel(dst_hbm, zeros_hbm, out_hbm, dst_v, ones_v, shared):
    c = lax.axis_index("c")
    s = lax.axis_index("s")
    wid = s * NC + c
    pltpu.sync_copy(zeros_hbm, shared.at[pl.ds(s * RPT, RPT)])
    one = jnp.ones((16,), jnp.float32)
    for i in range(CHUNK):
        for k in range(D // 16):
            ones_v[i, pl.ds(k * 16, 16)] = one
    pltpu.sync_copy(dst_hbm.at[wid], dst_v)
    plsc.subcore_barrier()

    def body(j, carry):
        pltpu.sync_copy(ones_v, shared.at[dst_v.at[j]], add=True)
        return carry

    lax.fori_loop(0, NCH, body, 0)
    plsc.subcore_barrier()
    pltpu.sync_copy(shared.at[pl.ds(s * RPT, RPT)],
                    out_hbm.at[c, pl.ds(s * RPT, RPT)])


@functools.partial(
    pl.kernel,
    out_type=jax.ShapeDtypeStruct((NC, NP, D), jnp.float32),
    mesh=_mesh,
    scratch_types=[
        pltpu.VMEM((EPW,), jnp.int32),
        pltpu.VMEM((2, CHUNK), jnp.int32),
        pltpu.VMEM((2, CHUNK), jnp.int32),
        pltpu.VMEM((CHUNK, D), jnp.float32),
        pltpu.VMEM((CHUNK, D), jnp.float32),
        pltpu.VMEM_SHARED((NP, D), jnp.float32),
        pltpu.SemaphoreType.DMA,
        pltpu.SemaphoreType.DMA,
        pltpu.SemaphoreType.DMA,
        pltpu.SemaphoreType.DMA,
    ],
)
def _acc_kernel(g_hbm, packed_hbm, zeros_hbm, out_hbm,
                packed_v, src_v, dst_v, rows0_v, rows1_v, shared,
                sem0, sem1, ssem0, ssem1):
    c = lax.axis_index("c")
    s = lax.axis_index("s")
    wid = s * NC + c
    pltpu.sync_copy(zeros_hbm, shared.at[pl.ds(s * RPT, RPT)])
    pltpu.sync_copy(packed_hbm.at[wid], packed_v)
    plsc.subcore_barrier()

    def unpack(j, k):
        for i in range(CHUNK // 16):
            v = packed_v[pl.ds(j * CHUNK + i * 16, 16)]
            src_v[k, pl.ds(i * 16, 16)] = lax.shift_right_logical(v, 14)
            dst_v[k, pl.ds(i * 16, 16)] = lax.bitwise_and(v, 16383)

    bufs = ((rows0_v, sem0, ssem0), (rows1_v, sem1, ssem1))
    unpack(0, 0)
    pltpu.async_copy(g_hbm.at[src_v.at[0]], rows0_v, sem0)

    def body(j, carry):
        for k in (0, 1):
            rows_v, sem, ssem = bufs[k]
            nrows_v, nsem, nssem = bufs[1 - k]

            @pl.when(lax.rem(j, 2) == k)
            def _():
                pltpu.make_async_copy(g_hbm.at[src_v.at[k]], rows_v, sem).wait()

                @pl.when(j + 1 < NCH)
                def _():
                    @pl.when(j > 0)
                    def _():
                        pltpu.make_async_copy(
                            nrows_v, shared.at[dst_v.at[1 - k]], nssem).wait()

                    unpack(j + 1, 1 - k)
                    pltpu.async_copy(g_hbm.at[src_v.at[1 - k]], nrows_v, nsem)

                pltpu.async_copy(rows_v, shared.at[dst_v.at[k]], ssem, add=True)

        return carry

    lax.fori_loop(0, NCH, body, 0)
    pltpu.make_async_copy(rows1_v, shared.at[dst_v.at[1]], ssem1).wait()
    pltpu.make_async_copy(rows0_v, shared.at[dst_v.at[0]], ssem0).wait()
    plsc.subcore_barrier()
    pltpu.sync_copy(shared.at[pl.ds(s * RPT, RPT)],
                    out_hbm.at[c, pl.ds(s * RPT, RPT)])


_BLK = 1000
_GRID = N // _BLK


def _tc_a_body(degp_ref, x_ref, w_ref, g_ref, dinv_ref):
    deg = degp_ref[0, :, 0:1] + degp_ref[1, :, 0:1] + 1.0
    dinv = lax.rsqrt(deg)
    dinv_ref[...] = dinv
    h = jnp.dot(x_ref[...], w_ref[...], preferred_element_type=jnp.float32)
    g_ref[...] = dinv * h


def _tc_a(degp, x, w):
    return pl.pallas_call(
        _tc_a_body,
        grid=(_GRID,),
        in_specs=[
            pl.BlockSpec((NC, _BLK, D), lambda i: (0, i, 0)),
            pl.BlockSpec((_BLK, D), lambda i: (i, 0)),
            pl.BlockSpec((D, D), lambda i: (0, 0)),
        ],
        out_specs=[
            pl.BlockSpec((_BLK, D), lambda i: (i, 0)),
            pl.BlockSpec((_BLK, 1), lambda i: (i, 0)),
        ],
        out_shape=[
            jax.ShapeDtypeStruct((N, D), jnp.float32),
            jax.ShapeDtypeStruct((N, 1), jnp.float32),
        ],
    )(degp, x, w)


def _tc_b_body(accp_ref, g_ref, dinv_ref, b_ref, a_ref, w_ref, g2_ref):
    dinv = dinv_ref[...]
    z = dinv * (accp_ref[0] + accp_ref[1] + g_ref[...]) + b_ref[...]
    z = jnp.where(z > 0, z, a_ref[...] * z)
    h = jnp.dot(z, w_ref[...], preferred_element_type=jnp.float32)
    g2_ref[...] = dinv * h


def _tc_b(accp, g, dinv, b, a, w):
    return pl.pallas_call(
        _tc_b_body,
        grid=(_GRID,),
        in_specs=[
            pl.BlockSpec((NC, _BLK, D), lambda i: (0, i, 0)),
            pl.BlockSpec((_BLK, D), lambda i: (i, 0)),
            pl.BlockSpec((_BLK, 1), lambda i: (i, 0)),
            pl.BlockSpec((1, D), lambda i: (0, 0)),
            pl.BlockSpec((1, D), lambda i: (0, 0)),
            pl.BlockSpec((D, D), lambda i: (0, 0)),
        ],
        out_specs=pl.BlockSpec((_BLK, D), lambda i: (i, 0)),
        out_shape=jax.ShapeDtypeStruct((N, D), jnp.float32),
    )(accp, g, dinv, b, a, w)


def _tc_c_body(accp_ref, g_ref, dinv_ref, b_ref, a_ref, out_ref):
    z = dinv_ref[...] * (accp_ref[0] + accp_ref[1] + g_ref[...]) + b_ref[...]
    out_ref[...] = jnp.where(z > 0, z, a_ref[...] * z)


def _tc_c(accp, g, dinv, b, a):
    return pl.pallas_call(
        _tc_c_body,
        grid=(_GRID,),
        in_specs=[
            pl.BlockSpec((NC, _BLK, D), lambda i: (0, i, 0)),
            pl.BlockSpec((_BLK, D), lambda i: (i, 0)),
            pl.BlockSpec((_BLK, 1), lambda i: (i, 0)),
            pl.BlockSpec((1, D), lambda i: (0, 0)),
            pl.BlockSpec((1, D), lambda i: (0, 0)),
        ],
        out_specs=pl.BlockSpec((_BLK, D), lambda i: (i, 0)),
        out_shape=jax.ShapeDtypeStruct((N, D), jnp.float32),
    )(accp, g, dinv, b, a)


def kernel(x, edge_index, W1, b1, a1, W2, b2, a2):
    ei = edge_index.astype(jnp.int32)
    dst3 = ei[1].reshape(NW, NCH, CHUNK)
    packed = ((ei[0] << 14) | ei[1]).reshape(NW, EPW)
    zerosD = jnp.zeros((RPT, D), jnp.float32)

    degp = _deg_kernel(dst3, zerosD)
    g1, dinv = _tc_a(degp, x, W1)
    accp1 = _acc_kernel(g1, packed, zerosD)
    g2 = _tc_b(accp1, g1, dinv, b1.reshape(1, D), a1.reshape(1, D), W2)
    accp2 = _acc_kernel(g2, packed, zerosD)
    return _tc_c(accp2, g2, dinv, b2.reshape(1, D), a2.reshape(1, D))


# speedup vs baseline: 1.3472x; 1.3472x over previous
import functools

import jax
import jax.numpy as jnp
from jax import lax
from jax.experimental import pallas as pl
from jax.experimental.pallas import tpu as pltpu
from jax.experimental.pallas import tpu_sc as plsc

N = 10000
E = 320000
D = 128
NC = 2
NS = 16
NW = NC * NS
EPW = E // NW
CHUNK = 80
NCH = EPW // CHUNK
NP = 10240
RPT = NP // NS

_mesh = plsc.VectorSubcoreMesh(core_axis_name="c", subcore_axis_name="s")


@functools.partial(
    pl.kernel,
    out_type=jax.ShapeDtypeStruct((NC, NP, D), jnp.float32),
    mesh=_mesh,
    scratch_types=[
        pltpu.VMEM((NCH, CHUNK), jnp.int32),
        pltpu.VMEM((CHUNK, D), jnp.float32),
        pltpu.VMEM_SHARED((NP, D), jnp.float32),
        pltpu.SemaphoreType.DMA,
    ],
)
def _deg_kernel(dst_hbm, zeros_hbm, out_hbm, dst_v, ones_v, shared, dsem):
    c = lax.axis_index("c")
    s = lax.axis_index("s")
    wid = s * NC + c
    pltpu.sync_copy(zeros_hbm, shared.at[pl.ds(s * RPT, RPT)])
    one = jnp.ones((16,), jnp.float32)
    for i in range(CHUNK):
        for k in range(D // 16):
            ones_v[i, pl.ds(k * 16, 16)] = one
    pltpu.sync_copy(dst_hbm.at[wid], dst_v)
    plsc.subcore_barrier()

    def body(j, carry):
        pltpu.async_copy(ones_v, shared.at[dst_v.at[j]], dsem, add=True)
        return carry

    lax.fori_loop(0, NCH, body, 0)

    def drain(j, carry):
        pltpu.make_async_copy(ones_v, shared.at[dst_v.at[0]], dsem).wait()
        return carry

    lax.fori_loop(0, NCH, drain, 0)
    plsc.subcore_barrier()
    pltpu.sync_copy(shared.at[pl.ds(s * RPT, RPT)],
                    out_hbm.at[c, pl.ds(s * RPT, RPT)])


@functools.partial(
    pl.kernel,
    out_type=jax.ShapeDtypeStruct((NC, NP, D), jnp.float32),
    mesh=_mesh,
    scratch_types=[
        pltpu.VMEM((EPW,), jnp.int32),
        pltpu.VMEM((3, CHUNK), jnp.int32),
        pltpu.VMEM((3, CHUNK), jnp.int32),
        pltpu.VMEM((CHUNK, D), jnp.float32),
        pltpu.VMEM((CHUNK, D), jnp.float32),
        pltpu.VMEM((CHUNK, D), jnp.float32),
        pltpu.VMEM_SHARED((NP, D), jnp.float32),
        pltpu.SemaphoreType.DMA,
        pltpu.SemaphoreType.DMA,
        pltpu.SemaphoreType.DMA,
        pltpu.SemaphoreType.DMA,
        pltpu.SemaphoreType.DMA,
        pltpu.SemaphoreType.DMA,
    ],
)
def _acc_kernel(g_hbm, packed_hbm, zeros_hbm, out_hbm,
                packed_v, src_v, dst_v, rows0_v, rows1_v, rows2_v, shared,
                sem0, sem1, sem2, ssem0, ssem1, ssem2):
    c = lax.axis_index("c")
    s = lax.axis_index("s")
    wid = s * NC + c
    pltpu.sync_copy(zeros_hbm, shared.at[pl.ds(s * RPT, RPT)])
    pltpu.sync_copy(packed_hbm.at[wid], packed_v)
    plsc.subcore_barrier()

    def unpack(j, k):
        for i in range(CHUNK // 16):
            v = packed_v[pl.ds(j * CHUNK + i * 16, 16)]
            src_v[k, pl.ds(i * 16, 16)] = lax.shift_right_logical(v, 14)
            dst_v[k, pl.ds(i * 16, 16)] = lax.bitwise_and(v, 16383)

    bufs = ((rows0_v, sem0, ssem0), (rows1_v, sem1, ssem1),
            (rows2_v, sem2, ssem2))
    unpack(0, 0)
    pltpu.async_copy(g_hbm.at[src_v.at[0]], rows0_v, sem0)
    unpack(1, 1)
    pltpu.async_copy(g_hbm.at[src_v.at[1]], rows1_v, sem1)

    def body(j, carry):
        for k in (0, 1, 2):
            rows_v, sem, ssem = bufs[k]
            nrows_v, nsem, nssem = bufs[(k + 2) % 3]

            @pl.when(lax.rem(j, 3) == k)
            def _():
                pltpu.make_async_copy(g_hbm.at[src_v.at[k]], rows_v, sem).wait()

                @pl.when(j + 2 < NCH)
                def _():
                    @pl.when(j > 0)
                    def _():
                        pltpu.make_async_copy(
                            nrows_v, shared.at[dst_v.at[(k + 2) % 3]],
                            nssem).wait()

                    unpack(j + 2, (k + 2) % 3)
                    pltpu.async_copy(g_hbm.at[src_v.at[(k + 2) % 3]],
                                     nrows_v, nsem)

                pltpu.async_copy(rows_v, shared.at[dst_v.at[k]], ssem, add=True)

        return carry

    lax.fori_loop(0, NCH, body, 0)
    for j in (NCH - 3, NCH - 2, NCH - 1):
        rows_v, _, ssem = bufs[j % 3]
        pltpu.make_async_copy(rows_v, shared.at[dst_v.at[j % 3]], ssem).wait()
    plsc.subcore_barrier()
    pltpu.sync_copy(shared.at[pl.ds(s * RPT, RPT)],
                    out_hbm.at[c, pl.ds(s * RPT, RPT)])


_BLK = 1000
_GRID = N // _BLK


def _tc_a_body(degp_ref, x_ref, w_ref, g_ref, dinv_ref):
    deg = degp_ref[0, :, 0:1] + degp_ref[1, :, 0:1] + 1.0
    dinv = lax.rsqrt(deg)
    dinv_ref[...] = dinv
    h = jnp.dot(x_ref[...], w_ref[...], preferred_element_type=jnp.float32)
    g_ref[...] = dinv * h


def _tc_a(degp, x, w):
    return pl.pallas_call(
        _tc_a_body,
        grid=(_GRID,),
        in_specs=[
            pl.BlockSpec((NC, _BLK, D), lambda i: (0, i, 0)),
            pl.BlockSpec((_BLK, D), lambda i: (i, 0)),
            pl.BlockSpec((D, D), lambda i: (0, 0)),
        ],
        out_specs=[
            pl.BlockSpec((_BLK, D), lambda i: (i, 0)),
            pl.BlockSpec((_BLK, 1), lambda i: (i, 0)),
        ],
        out_shape=[
            jax.ShapeDtypeStruct((N, D), jnp.float32),
            jax.ShapeDtypeStruct((N, 1), jnp.float32),
        ],
    )(degp, x, w)


def _tc_b_body(accp_ref, g_ref, dinv_ref, b_ref, a_ref, w_ref, g2_ref):
    dinv = dinv_ref[...]
    z = dinv * (accp_ref[0] + accp_ref[1] + g_ref[...]) + b_ref[...]
    z = jnp.where(z > 0, z, a_ref[...] * z)
    h = jnp.dot(z, w_ref[...], preferred_element_type=jnp.float32)
    g2_ref[...] = dinv * h


def _tc_b(accp, g, dinv, b, a, w):
    return pl.pallas_call(
        _tc_b_body,
        grid=(_GRID,),
        in_specs=[
            pl.BlockSpec((NC, _BLK, D), lambda i: (0, i, 0)),
            pl.BlockSpec((_BLK, D), lambda i: (i, 0)),
            pl.BlockSpec((_BLK, 1), lambda i: (i, 0)),
            pl.BlockSpec((1, D), lambda i: (0, 0)),
            pl.BlockSpec((1, D), lambda i: (0, 0)),
            pl.BlockSpec((D, D), lambda i: (0, 0)),
        ],
        out_specs=pl.BlockSpec((_BLK, D), lambda i: (i, 0)),
        out_shape=jax.ShapeDtypeStruct((N, D), jnp.float32),
    )(accp, g, dinv, b, a, w)


def _tc_c_body(accp_ref, g_ref, dinv_ref, b_ref, a_ref, out_ref):
    z = dinv_ref[...] * (accp_ref[0] + accp_ref[1] + g_ref[...]) + b_ref[...]
    out_ref[...] = jnp.where(z > 0, z, a_ref[...] * z)


def _tc_c(accp, g, dinv, b, a):
    return pl.pallas_call(
        _tc_c_body,
        grid=(_GRID,),
        in_specs=[
            pl.BlockSpec((NC, _BLK, D), lambda i: (0, i, 0)),
            pl.BlockSpec((_BLK, D), lambda i: (i, 0)),
            pl.BlockSpec((_BLK, 1), lambda i: (i, 0)),
            pl.BlockSpec((1, D), lambda i: (0, 0)),
            pl.BlockSpec((1, D), lambda i: (0, 0)),
        ],
        out_specs=pl.BlockSpec((_BLK, D), lambda i: (i, 0)),
        out_shape=jax.ShapeDtypeStruct((N, D), jnp.float32),
    )(accp, g, dinv, b, a)


def kernel(x, edge_index, W1, b1, a1, W2, b2, a2):
    ei = edge_index.astype(jnp.int32)
    dst3 = ei[1].reshape(NW, NCH, CHUNK)
    packed = ((ei[0] << 14) | ei[1]).reshape(NW, EPW)
    zerosD = jnp.zeros((RPT, D), jnp.float32)

    degp = _deg_kernel(dst3, zerosD)
    g1, dinv = _tc_a(degp, x, W1)
    accp1 = _acc_kernel(g1, packed, zerosD)
    g2 = _tc_b(accp1, g1, dinv, b1.reshape(1, D), a1.reshape(1, D), W2)
    accp2 = _acc_kernel(g2, packed, zerosD)
    return _tc_c(accp2, g2, dinv, b2.reshape(1, D), a2.reshape(1, D))
